# baseline (speedup 1.0000x reference)
import functools

import jax
import jax.numpy as jnp
from jax import lax
from jax.experimental import pallas as pl
from jax.experimental.pallas import tpu as pltpu

N = 262144
B = 8
G = 32
NSEG = B * G * G * G
C_PT = 128
H1 = 256
C_OUT = 128
EPS = 1e-5

_BN = 2048
_INTERPRET = False


def _pass_a_body(nocs_ref, xyz_ref, bidx_ref, ppf_ref, w1a_ref, w1b_ref, b1_ref,
                 flat_ref, extra_ref, s1_ref, ss1_ref):
    i = pl.program_id(0)
    nocs = nocs_ref[...]
    gs1 = jnp.float32(G - 1)
    idx_f = jnp.clip(jnp.round(nocs * gs1), 0.0, gs1)
    idx = idx_f.astype(jnp.int32)
    bidx = bidx_ref[...]
    flat = (bidx[:, 0] * (G * G * G)
            + idx[:, 0] * (G * G) + idx[:, 1] * G + idx[:, 2])
    flat_ref[...] = flat[:, None]
    grid_pts = idx_f * (1.0 / gs1)
    lo = nocs - grid_pts
    xyz = xyz_ref[...]
    zeros2 = jnp.zeros((lo.shape[0], 2), jnp.float32)
    extra = jnp.concatenate([lo, xyz, zeros2], axis=1)
    extra_ref[...] = extra
    p1 = (jnp.dot(ppf_ref[...], w1a_ref[...], preferred_element_type=jnp.float32)
          + jnp.dot(extra, w1b_ref[...], preferred_element_type=jnp.float32)
          + b1_ref[...])
    h = jnp.maximum(p1, 0.0)

    @pl.when(i == 0)
    def _():
        s1_ref[...] = jnp.zeros_like(s1_ref)
        ss1_ref[...] = jnp.zeros_like(ss1_ref)

    s1_ref[...] += jnp.sum(h, axis=0, keepdims=True)
    ss1_ref[...] += jnp.sum(h * h, axis=0, keepdims=True)


def _pass_b_body(ppf_ref, extra_ref, w1a_ref, w1b_ref, b1_ref,
                 s1_ref, ss1_ref, g1_ref, bt1_ref, w2_ref, b2_ref,
                 h2_ref, s2_ref, ss2_ref):
    i = pl.program_id(0)
    mu1 = s1_ref[...] * (1.0 / N)
    var1 = ss1_ref[...] * (1.0 / N) - mu1 * mu1
    a1 = g1_ref[...] * lax.rsqrt(var1 + EPS)
    c1 = bt1_ref[...] - mu1 * a1
    p1 = (jnp.dot(ppf_ref[...], w1a_ref[...], preferred_element_type=jnp.float32)
          + jnp.dot(extra_ref[...], w1b_ref[...], preferred_element_type=jnp.float32)
          + b1_ref[...])
    h1 = jnp.maximum(p1, 0.0)
    h1s = h1 * a1
    p2 = (jnp.dot(h1s, w2_ref[...], preferred_element_type=jnp.float32)
          + jnp.dot(c1, w2_ref[...], preferred_element_type=jnp.float32)
          + b2_ref[...])
    h2 = jnp.maximum(p2, 0.0)
    h2_ref[...] = h2

    @pl.when(i == 0)
    def _():
        s2_ref[...] = jnp.zeros_like(s2_ref)
        ss2_ref[...] = jnp.zeros_like(ss2_ref)

    s2_ref[...] += jnp.sum(h2, axis=0, keepdims=True)
    ss2_ref[...] += jnp.sum(h2 * h2, axis=0, keepdims=True)


def _pass_d_body(sums_ref, cnt_ref, s2_ref, ss2_ref, g2_ref, bt2_ref, out_ref):
    mu2 = s2_ref[...] * (1.0 / N)
    var2 = ss2_ref[...] * (1.0 / N) - mu2 * mu2
    a2 = g2_ref[...] * lax.rsqrt(var2 + EPS)
    c2 = bt2_ref[...] - mu2 * a2
    cnt = cnt_ref[...]
    mean = sums_ref[...] / jnp.maximum(cnt, 1.0)
    o = mean * a2 + jnp.where(cnt > 0.0, c2, 0.0)
    out_ref[...] = jnp.transpose(o)[None]


def _segment_sums(flat, h2):
    sums = jax.ops.segment_sum(h2, flat, num_segments=NSEG)
    counts = jax.ops.segment_sum(jnp.ones((N, 1), jnp.float32), flat,
                                 num_segments=NSEG)
    return sums, counts


def kernel(xyz, pred_nocs, per_point_features, pred_confidence, batch_size,
           batch_idx, W1, b1, g1, bt1, W2, b2, g2, bt2):
    del pred_confidence, batch_size
    bidx2 = batch_idx.astype(jnp.int32).reshape(N, 1)
    w1a = W1[:C_PT]
    w1b = jnp.concatenate([W1[C_PT:], jnp.zeros((2, H1), jnp.float32)], axis=0)
    b1r = b1.reshape(1, H1)
    g1r = g1.reshape(1, H1)
    bt1r = bt1.reshape(1, H1)
    b2r = b2.reshape(1, C_OUT)
    g2r = g2.reshape(1, C_OUT)
    bt2r = bt2.reshape(1, C_OUT)

    nsteps = N // _BN
    row_spec = lambda w: pl.BlockSpec((_BN, w), lambda i: (i, 0))
    full_spec = lambda a, b: pl.BlockSpec((a, b), lambda i: (0, 0))

    flat, extra, s1, ss1 = pl.pallas_call(
        _pass_a_body,
        grid=(nsteps,),
        in_specs=[row_spec(3), row_spec(3), row_spec(1), row_spec(C_PT),
                  full_spec(C_PT, H1), full_spec(8, H1), full_spec(1, H1)],
        out_specs=[row_spec(1), row_spec(8), full_spec(1, H1), full_spec(1, H1)],
        out_shape=[jax.ShapeDtypeStruct((N, 1), jnp.int32),
                   jax.ShapeDtypeStruct((N, 8), jnp.float32),
                   jax.ShapeDtypeStruct((1, H1), jnp.float32),
                   jax.ShapeDtypeStruct((1, H1), jnp.float32)],
        compiler_params=pltpu.CompilerParams(
            dimension_semantics=("arbitrary",)),
        interpret=_INTERPRET,
    )(pred_nocs, xyz, bidx2, per_point_features, w1a, w1b, b1r)

    h2, s2, ss2 = pl.pallas_call(
        _pass_b_body,
        grid=(nsteps,),
        in_specs=[row_spec(C_PT), row_spec(8),
                  full_spec(C_PT, H1), full_spec(8, H1), full_spec(1, H1),
                  full_spec(1, H1), full_spec(1, H1), full_spec(1, H1),
                  full_spec(1, H1), full_spec(H1, C_OUT), full_spec(1, C_OUT)],
        out_specs=[row_spec(C_OUT), full_spec(1, C_OUT), full_spec(1, C_OUT)],
        out_shape=[jax.ShapeDtypeStruct((N, C_OUT), jnp.float32),
                   jax.ShapeDtypeStruct((1, C_OUT), jnp.float32),
                   jax.ShapeDtypeStruct((1, C_OUT), jnp.float32)],
        compiler_params=pltpu.CompilerParams(
            dimension_semantics=("arbitrary",)),
        interpret=_INTERPRET,
    )(per_point_features, extra, w1a, w1b, b1r, s1, ss1, g1r, bt1r, W2, b2r)

    sums, counts = _segment_sums(flat[:, 0], h2)

    bS = 2048
    dsteps = NSEG // bS
    per_b = (G * G * G) // bS
    out = pl.pallas_call(
        _pass_d_body,
        grid=(dsteps,),
        in_specs=[pl.BlockSpec((bS, C_OUT), lambda i: (i, 0)),
                  pl.BlockSpec((bS, 1), lambda i: (i, 0)),
                  full_spec(1, C_OUT), full_spec(1, C_OUT),
                  full_spec(1, C_OUT), full_spec(1, C_OUT)],
        out_specs=pl.BlockSpec((1, C_OUT, bS),
                               lambda i: (i // per_b, 0, i % per_b)),
        out_shape=jax.ShapeDtypeStruct((B, C_OUT, G * G * G), jnp.float32),
        compiler_params=pltpu.CompilerParams(
            dimension_semantics=("arbitrary",)),
        interpret=_INTERPRET,
    )(sums, counts, s2, ss2, g2r, bt2r)

    return out.reshape(B, C_OUT, G, G, G)
